# baseline (speedup 1.0000x reference)
import functools

import jax
import jax.numpy as jnp
from jax import lax
from jax.experimental import pallas as pl
from jax.experimental.pallas import tpu as pltpu
from jax.experimental.pallas import tpu_sc as plsc

G = 3
N = 50000
NP1 = N + 1
D = 128
E = 500000
EP = E + 512
B = 4096
LEAKY = 0.5

NC = 2
NS = 16
L = 16

C = 3200
NCHUNK = 16
CT = C // NS
RB = 40
K = 128
KH = 2
K2 = K * KH
SLOTS = 16
TRASH = NS * 128


def _iota16():
    return lax.iota(jnp.int32, L)


def _sget(ref, i):
    return ref[pl.ds(i, L)][0]


def _seg_pass(src_tab, tgt_arr, src_arr, rs, base_tab, out_tab,
              idxb, tgtb, sidxb, gixb, rowsb, epi_t, epi_b, zbuf, bnds,
              acc, semi, semt, semg, sems_, cid, sid):
    def _wait_idx(p):
        for h in range(KH):
            pltpu.make_async_copy(src_arr.at[pl.ds(0, K)], idxb.at[p, h],
                                  semi.at[p]).wait()
            pltpu.make_async_copy(tgt_arr.at[pl.ds(0, K)], tgtb.at[p, h],
                                  semt.at[p]).wait()

    def _wait_gather(p):
        pltpu.make_async_copy(src_tab.at[pl.ds(0, K2)], rowsb.at[p], semg.at[p]).wait()

    def _wait_scat(p):
        pltpu.make_async_copy(rowsb.at[p], acc.at[pl.ds(0, K2)], sems_.at[p]).wait()

    def _issue_gather(p):
        for h in range(KH):
            pltpu.async_copy(src_tab.at[gixb.at[p, h]],
                             rowsb.at[p, pl.ds(h * K, K)], semg.at[p])

    def _issue_scatter(p):
        for h in range(KH):
            pltpu.async_copy(rowsb.at[p, pl.ds(h * K, K)],
                             acc.at[sidxb.at[p, h]], sems_.at[p], add=True)

    @pl.loop(0, G)
    def _graph(k):
        koff_tab = k * N
        koff_e = k * EP
        koff_rs = k * NP1

        bidx = jnp.minimum(_iota16() * C, N) + koff_rs
        pltpu.sync_copy(rs.at[bidx], bnds.at[pl.ds(0, L)])
        bidx2 = jnp.minimum((_iota16() + L) * C, N) + koff_rs
        pltpu.sync_copy(rs.at[bidx2], bnds.at[pl.ds(L, L)])

        @pl.loop(0, NCHUNK // NC)
        def _chunk(j):
            c = NC * j + cid
            nlo = c * C
            nhi = jnp.minimum(nlo + C, N)

            @pl.loop(0, CT // RB)
            def _zero(r):
                pltpu.sync_copy(zbuf, acc.at[pl.ds(sid * CT + r * RB, RB)])

            plsc.subcore_barrier()

            lo = _sget(bnds, c)
            hi = _sget(bnds, c + 1)
            ln = hi - lo
            el = lo + (ln * sid) // NS
            eh = lo + (ln * (sid + 1)) // NS
            el8 = (el // 8) * 8
            nsteps = (eh - el8 + K2 - 1) // K2

            def _fetch(t, p):
                base = el8 + t * K2
                for h in range(KH):
                    pltpu.async_copy(src_arr.at[pl.ds(koff_e + base + h * K, K)],
                                     idxb.at[p, h], semi.at[p])
                    pltpu.async_copy(tgt_arr.at[pl.ds(koff_e + base + h * K, K)],
                                     tgtb.at[p, h], semt.at[p])

            @pl.when(nsteps > 0)
            def _prologue():
                _fetch(0, 0)

            @pl.loop(0, nsteps)
            def _edges(t):
                p = t % 2
                q = 1 - p
                base = el8 + t * K2
                _wait_idx(p)

                @pl.when(t + 1 < nsteps)
                def _next():
                    _fetch(t + 1, q)

                for g in range(K2 // L):
                    h = g // (K // L)
                    gl = g % (K // L)
                    tv = tgtb[p, h, pl.ds(gl * L, L)]
                    iv = idxb[p, h, pl.ds(gl * L, L)]
                    pos = base + g * L + _iota16()
                    m = (pos >= el) & (pos < eh)
                    sidxb[p, h, pl.ds(gl * L, L)] = jnp.where(m, tv - nlo, C)
                    gixb[p, h, pl.ds(gl * L, L)] = iv + koff_tab

                @pl.when(t >= 2)
                def _reuse():
                    _wait_scat(p)

                _issue_gather(p)
                _wait_gather(p)
                _issue_scatter(p)

            @pl.when(nsteps >= 2)
            def _d2():
                _wait_scat(nsteps % 2)

            @pl.when(nsteps >= 1)
            def _d1():
                _wait_scat((nsteps - 1) % 2)

            plsc.subcore_barrier()

            rl = nlo + sid * CT
            cnt = jnp.minimum(rl + CT, nhi) - rl

            @pl.loop(0, (cnt + RB - 1) // RB)
            def _epi(r):
                off = r * RB
                pltpu.sync_copy(acc.at[pl.ds(sid * CT + off, RB)], epi_t)
                pltpu.sync_copy(base_tab.at[pl.ds(koff_tab + rl + off, RB)], epi_b)

                @pl.loop(0, RB)
                def _row(rr):
                    for g in range(D // L):
                        t = epi_t[rr, pl.ds(g * L, L)]
                        bse = epi_b[rr, pl.ds(g * L, L)]
                        epi_t[rr, pl.ds(g * L, L)] = jnp.maximum(t, LEAKY * t) + bse

                pltpu.sync_copy(epi_t, out_tab.at[pl.ds(koff_tab + rl + off, RB)])

            plsc.subcore_barrier()


def _layer1_body(U0f, I0f, adjt, adjs, tpt, tps, rsa, rst, U1f, I1f,
                 idxb, tgtb, sidxb, gixb, rowsb, epi_t, epi_b, zbuf, bnds, acc,
                 semi, semt, semg, sems_):
    cid = lax.axis_index("c")
    sid = lax.axis_index("s")

    @pl.loop(0, RB)
    def _z(rr):
        for g in range(D // L):
            zbuf[rr, pl.ds(g * L, L)] = jnp.zeros((L,), jnp.float32)

    _seg_pass(I0f, adjt, adjs, rsa, U0f, U1f,
              idxb, tgtb, sidxb, gixb, rowsb, epi_t, epi_b, zbuf, bnds, acc,
              semi, semt, semg, sems_, cid, sid)
    _seg_pass(U0f, tpt, tps, rst, I0f, I1f,
              idxb, tgtb, sidxb, gixb, rowsb, epi_t, epi_b, zbuf, bnds, acc,
              semi, semt, semg, sems_, cid, sid)


def _demand_side(k, tab0, tab1, gtab, rs, earr, ids, accbuf,
                 gidx, sb_idx, sbuf, ebuf, sv, sv16, gibuf, posb, ridxb,
                 r0, r1, rows16, zb, a2, sid):
    koff_tab = k * N
    koff_e = k * EP
    koff_rs = k * NP1

    for g in range(128 // L):
        iv = ids[pl.ds(g * L, L)]
        gidx[pl.ds(g * L, L)] = iv + koff_tab
        sb_idx[pl.ds(g * L, L)] = iv + koff_rs
    pltpu.sync_copy(rs.at[sb_idx], sbuf.at[pl.ds(0, 128)])
    for g in range(128 // L):
        sb_idx[pl.ds(g * L, L)] = sb_idx[pl.ds(g * L, L)] + 1
    pltpu.sync_copy(rs.at[sb_idx], ebuf.at[pl.ds(0, 128)])

    pltpu.sync_copy(tab0.at[gidx], r0)
    pltpu.sync_copy(tab1.at[gidx], r1)

    @pl.loop(0, 128)
    def _accrows(b):
        for g in range(D // L):
            accbuf[b, pl.ds(g * L, L)] = (accbuf[b, pl.ds(g * L, L)]
                                          + r0[b, pl.ds(g * L, L)]
                                          + 2.0 * r1[b, pl.ds(g * L, L)])

    pltpu.sync_copy(zb, a2.at[pl.ds(sid * 128, 128)])

    @pl.loop(0, 128)
    def _build(b):
        s = _sget(sbuf, b)
        e = _sget(ebuf, b)
        g = b // 8
        col0 = (b % 8) * L
        m = _iota16() < (e - s)
        posb[g, pl.ds(col0, L)] = koff_e + s + _iota16()
        ridxb[g, pl.ds(col0, L)] = jnp.where(m, sid * 128 + b, TRASH)

    for g in range(SLOTS):
        pltpu.sync_copy(earr.at[posb.at[g]], sv)
        for gg in range(128 // L):
            gibuf[pl.ds(gg * L, L)] = sv[pl.ds(gg * L, L)] + koff_tab
        pltpu.sync_copy(gtab.at[gibuf], r0)
        pltpu.sync_copy(r0, a2.at[ridxb.at[g]], add=True)

    @pl.loop(0, 128)
    def _ovf(b):
        s = _sget(sbuf, b)
        e = _sget(ebuf, b)
        nov = (e - s - SLOTS + L - 1) // L

        @pl.loop(0, nov)
        def _chunk(t):
            base = s + SLOTS + t * L
            posv = koff_e + base + _iota16()
            m = (base + _iota16()) < e
            pltpu.sync_copy(earr.at[posv], sv16)
            giv = sv16[pl.ds(0, L)] + koff_tab
            pltpu.sync_copy(gtab.at[giv], rows16)
            ridxv = jnp.where(m, sid * 128 + b, TRASH)
            pltpu.sync_copy(rows16, a2.at[ridxv], add=True)

    pltpu.sync_copy(a2.at[pl.ds(sid * 128, 128)], r1)

    @pl.loop(0, 128)
    def _act(b):
        for g in range(D // L):
            v = r1[b, pl.ds(g * L, L)]
            accbuf[b, pl.ds(g * L, L)] = (accbuf[b, pl.ds(g * L, L)]
                                          + jnp.maximum(v, LEAKY * v))


def _final_body(U0f, I0f, U1f, I1f, adjs, tps, rsa, rst, uids, iids, preds,
                uv, ivv, gidx, sb_idx, sbuf, ebuf, sv, sv16, gibuf, posb, ridxb,
                r0, r1, rows16, fu, fi, zb, pb, a2):
    cid = lax.axis_index("c")
    sid = lax.axis_index("s")
    wloc = cid * NS + sid

    pltpu.sync_copy(uids.at[pl.ds(wloc * 128, 128)], uv)
    pltpu.sync_copy(iids.at[pl.ds(wloc * 128, 128)], ivv)

    @pl.loop(0, 128)
    def _zero(b):
        for g in range(D // L):
            fu[b, pl.ds(g * L, L)] = jnp.zeros((L,), jnp.float32)
            fi[b, pl.ds(g * L, L)] = jnp.zeros((L,), jnp.float32)
            zb[b, pl.ds(g * L, L)] = jnp.zeros((L,), jnp.float32)

    @pl.loop(0, G)
    def _graph(k):
        _demand_side(k, U0f, U1f, I1f, rsa, adjs, uv, fu,
                     gidx, sb_idx, sbuf, ebuf, sv, sv16, gibuf, posb, ridxb,
                     r0, r1, rows16, zb, a2, sid)
        _demand_side(k, I0f, I1f, U1f, rst, tps, ivv, fi,
                     gidx, sb_idx, sbuf, ebuf, sv, sv16, gibuf, posb, ridxb,
                     r0, r1, rows16, zb, a2, sid)

    @pl.loop(0, 128 // L)
    def _dot(bb):
        out = jnp.zeros((L,), jnp.float32)
        for j in range(L):
            b = bb * L + j
            part = fu[b, pl.ds(0, L)] * fi[b, pl.ds(0, L)]
            for g in range(1, D // L):
                part = part + fu[b, pl.ds(g * L, L)] * fi[b, pl.ds(g * L, L)]
            for s in (8, 4, 2, 1):
                part = part + jnp.take_along_axis(part, (_iota16() + s) % L, axis=0)
            out = jnp.where(_iota16() == j, part * (1.0 / 9.0), out)
        pb[pl.ds(bb * L, L)] = out

    pltpu.sync_copy(pb, preds.at[pl.ds(wloc * 128, 128)])


def kernel(user_embeddings, item_embeddings, adj_tgt, adj_src,
           tpadj_tgt, tpadj_src, uids, iids):
    f32 = jnp.float32
    U0f = user_embeddings.reshape(G * N, D)
    I0f = item_embeddings.reshape(G * N, D)

    adj_tgt = adj_tgt.astype(jnp.int32)
    adj_src = adj_src.astype(jnp.int32)
    tpadj_tgt = tpadj_tgt.astype(jnp.int32)
    tpadj_src = tpadj_src.astype(jnp.int32)
    uids32 = uids.astype(jnp.int32)
    iids32 = iids.astype(jnp.int32)

    def _offsets(tgt):
        deg = jnp.zeros((N,), jnp.int32).at[tgt].add(1, mode="drop")
        return jnp.concatenate([jnp.zeros((1,), jnp.int32), jnp.cumsum(deg)])

    rsa_f = jax.vmap(_offsets)(adj_tgt).astype(jnp.int32).reshape(-1)
    rst_f = jax.vmap(_offsets)(tpadj_tgt).astype(jnp.int32).reshape(-1)

    pad = ((0, 0), (0, EP - E))
    adjt_f = jnp.pad(adj_tgt, pad).reshape(-1)
    adjs_f = jnp.pad(adj_src, pad).reshape(-1)
    tpt_f = jnp.pad(tpadj_tgt, pad).reshape(-1)
    tps_f = jnp.pad(tpadj_src, pad).reshape(-1)

    mesh = plsc.VectorSubcoreMesh(core_axis_name="c", subcore_axis_name="s",
                                  num_cores=NC, num_subcores=NS)

    layer1 = pl.kernel(
        _layer1_body,
        out_type=[jax.ShapeDtypeStruct((G * N, D), f32),
                  jax.ShapeDtypeStruct((G * N, D), f32)],
        mesh=mesh,
        scratch_types=[
            pltpu.VMEM((2, KH, K), jnp.int32),
            pltpu.VMEM((2, KH, K), jnp.int32),
            pltpu.VMEM((2, KH, K), jnp.int32),
            pltpu.VMEM((2, KH, K), jnp.int32),
            pltpu.VMEM((2, K2, D), f32),
            pltpu.VMEM((RB, D), f32),
            pltpu.VMEM((RB, D), f32),
            pltpu.VMEM((RB, D), f32),
            pltpu.VMEM((2 * L,), jnp.int32),
            pltpu.VMEM_SHARED((C + 8, D), f32),
            pltpu.SemaphoreType.DMA((2,)),
            pltpu.SemaphoreType.DMA((2,)),
            pltpu.SemaphoreType.DMA((2,)),
            pltpu.SemaphoreType.DMA((2,)),
        ],
    )
    U1f, I1f = layer1(U0f, I0f, adjt_f, adjs_f, tpt_f, tps_f, rsa_f, rst_f)

    final = pl.kernel(
        _final_body,
        out_type=jax.ShapeDtypeStruct((B,), f32),
        mesh=mesh,
        scratch_types=[
            pltpu.VMEM((128,), jnp.int32),
            pltpu.VMEM((128,), jnp.int32),
            pltpu.VMEM((128,), jnp.int32),
            pltpu.VMEM((128,), jnp.int32),
            pltpu.VMEM((144,), jnp.int32),
            pltpu.VMEM((144,), jnp.int32),
            pltpu.VMEM((128,), jnp.int32),
            pltpu.VMEM((L,), jnp.int32),
            pltpu.VMEM((128,), jnp.int32),
            pltpu.VMEM((SLOTS, 128), jnp.int32),
            pltpu.VMEM((SLOTS, 128), jnp.int32),
            pltpu.VMEM((128, D), f32),
            pltpu.VMEM((128, D), f32),
            pltpu.VMEM((L, D), f32),
            pltpu.VMEM((128, D), f32),
            pltpu.VMEM((128, D), f32),
            pltpu.VMEM((128, D), f32),
            pltpu.VMEM((128,), f32),
            pltpu.VMEM_SHARED((TRASH + 8, D), f32),
        ],
    )
    return final(U0f, I0f, U1f, I1f, adjs_f, tps_f, rsa_f, rst_f,
                 uids32, iids32)


# speedup vs baseline: 1.0341x; 1.0341x over previous
import functools

import jax
import jax.numpy as jnp
from jax import lax
from jax.experimental import pallas as pl
from jax.experimental.pallas import tpu as pltpu
from jax.experimental.pallas import tpu_sc as plsc

G = 3
N = 50000
NP1 = N + 1
D = 128
E = 500000
EP = E + 512
B = 4096
LEAKY = 0.5

NC = 2
NS = 16
L = 16

C = 3200
NCHUNK = 16
CT = C // NS
RB = 40
K = 128
KH = 2
K2 = K * KH
SLOTS = 16
TRASH = NS * 128


def _iota16():
    return lax.iota(jnp.int32, L)


def _sget(ref, i):
    return ref[pl.ds(i, L)][0]


def _seg_pass(src_tab, tgt_arr, src_arr, rs, base_tab, out_tab,
              idxb, tgtb, sidxb, gixb, rowsb, epi_t, epi_b, zbuf, bnds,
              acc, semi, semt, semg, sems_, cid, sid):
    def _wait_idx(p):
        for h in range(KH):
            pltpu.make_async_copy(src_arr.at[pl.ds(0, K)], idxb.at[p, h],
                                  semi.at[p]).wait()
            pltpu.make_async_copy(tgt_arr.at[pl.ds(0, K)], tgtb.at[p, h],
                                  semt.at[p]).wait()

    def _wait_gather(p):
        pltpu.make_async_copy(src_tab.at[pl.ds(0, K2)], rowsb.at[p], semg.at[p]).wait()

    def _wait_scat(p):
        pltpu.make_async_copy(rowsb.at[p], acc.at[pl.ds(0, K2)], sems_.at[p]).wait()

    def _issue_gather(p):
        for h in range(KH):
            pltpu.async_copy(src_tab.at[gixb.at[p, h]],
                             rowsb.at[p, pl.ds(h * K, K)], semg.at[p])

    def _issue_scatter(p):
        for h in range(KH):
            pltpu.async_copy(rowsb.at[p, pl.ds(h * K, K)],
                             acc.at[sidxb.at[p, h]], sems_.at[p], add=True)

    @pl.loop(0, G)
    def _graph(k):
        koff_tab = k * N
        koff_e = k * EP
        koff_rs = k * NP1

        bidx = jnp.minimum(_iota16() * C, N) + koff_rs
        pltpu.sync_copy(rs.at[bidx], bnds.at[pl.ds(0, L)])
        bidx2 = jnp.minimum((_iota16() + L) * C, N) + koff_rs
        pltpu.sync_copy(rs.at[bidx2], bnds.at[pl.ds(L, L)])

        @pl.loop(0, NCHUNK // NC)
        def _chunk(j):
            c = NC * j + cid
            nlo = c * C
            nhi = jnp.minimum(nlo + C, N)

            @pl.loop(0, CT // RB)
            def _zero(r):
                pltpu.sync_copy(zbuf, acc.at[pl.ds(sid * CT + r * RB, RB)])

            plsc.subcore_barrier()

            lo = _sget(bnds, c)
            hi = _sget(bnds, c + 1)
            ln = hi - lo
            el = lo + (ln * sid) // NS
            eh = lo + (ln * (sid + 1)) // NS
            el8 = (el // 8) * 8
            nsteps = (eh - el8 + K2 - 1) // K2

            def _fetch(t, p):
                base = el8 + t * K2
                for h in range(KH):
                    pltpu.async_copy(src_arr.at[pl.ds(koff_e + base + h * K, K)],
                                     idxb.at[p, h], semi.at[p])
                    pltpu.async_copy(tgt_arr.at[pl.ds(koff_e + base + h * K, K)],
                                     tgtb.at[p, h], semt.at[p])

            @pl.when(nsteps > 0)
            def _prologue():
                _fetch(0, 0)

            @pl.loop(0, nsteps)
            def _edges(t):
                p = t % 2
                q = 1 - p
                base = el8 + t * K2
                _wait_idx(p)

                @pl.when(t + 1 < nsteps)
                def _next():
                    _fetch(t + 1, q)

                @pl.when(t >= 2)
                def _reuse2():
                    _wait_scat(p)

                for g in range(K2 // L):
                    h = g // (K // L)
                    gl = g % (K // L)
                    tv = tgtb[p, h, pl.ds(gl * L, L)]
                    iv = idxb[p, h, pl.ds(gl * L, L)]
                    pos = base + g * L + _iota16()
                    m = (pos >= el) & (pos < eh)
                    sidxb[p, h, pl.ds(gl * L, L)] = jnp.where(m, tv - nlo, C)
                    gixb[p, h, pl.ds(gl * L, L)] = iv + koff_tab

                _issue_gather(p)

                @pl.when(t >= 1)
                def _pipe():
                    _wait_gather(q)
                    _issue_scatter(q)

            @pl.when(nsteps >= 1)
            def _tail():
                pl_ = (nsteps - 1) % 2
                _wait_gather(pl_)
                _issue_scatter(pl_)

                @pl.when(nsteps >= 2)
                def _d2():
                    _wait_scat(nsteps % 2)

                _wait_scat(pl_)

            plsc.subcore_barrier()

            rl = nlo + sid * CT
            cnt = jnp.minimum(rl + CT, nhi) - rl

            @pl.loop(0, (cnt + RB - 1) // RB)
            def _epi(r):
                off = r * RB
                pltpu.sync_copy(acc.at[pl.ds(sid * CT + off, RB)], epi_t)
                pltpu.sync_copy(base_tab.at[pl.ds(koff_tab + rl + off, RB)], epi_b)

                @pl.loop(0, RB)
                def _row(rr):
                    for g in range(D // L):
                        t = epi_t[rr, pl.ds(g * L, L)]
                        bse = epi_b[rr, pl.ds(g * L, L)]
                        epi_t[rr, pl.ds(g * L, L)] = jnp.maximum(t, LEAKY * t) + bse

                pltpu.sync_copy(epi_t, out_tab.at[pl.ds(koff_tab + rl + off, RB)])

            plsc.subcore_barrier()


def _layer1_body(U0f, I0f, adjt, adjs, tpt, tps, rsa, rst, U1f, I1f,
                 idxb, tgtb, sidxb, gixb, rowsb, epi_t, epi_b, zbuf, bnds, acc,
                 semi, semt, semg, sems_):
    cid = lax.axis_index("c")
    sid = lax.axis_index("s")

    @pl.loop(0, RB)
    def _z(rr):
        for g in range(D // L):
            zbuf[rr, pl.ds(g * L, L)] = jnp.zeros((L,), jnp.float32)

    _seg_pass(I0f, adjt, adjs, rsa, U0f, U1f,
              idxb, tgtb, sidxb, gixb, rowsb, epi_t, epi_b, zbuf, bnds, acc,
              semi, semt, semg, sems_, cid, sid)
    _seg_pass(U0f, tpt, tps, rst, I0f, I1f,
              idxb, tgtb, sidxb, gixb, rowsb, epi_t, epi_b, zbuf, bnds, acc,
              semi, semt, semg, sems_, cid, sid)


def _demand_side(k, tab0, tab1, gtab, rs, earr, ids, accbuf,
                 gidx, sb_idx, sbuf, ebuf, sv, sv16, gibuf, posb, ridxb,
                 r0, r1, rows16, zb, a2, sid):
    koff_tab = k * N
    koff_e = k * EP
    koff_rs = k * NP1

    for g in range(128 // L):
        iv = ids[pl.ds(g * L, L)]
        gidx[pl.ds(g * L, L)] = iv + koff_tab
        sb_idx[pl.ds(g * L, L)] = iv + koff_rs
    pltpu.sync_copy(rs.at[sb_idx], sbuf.at[pl.ds(0, 128)])
    for g in range(128 // L):
        sb_idx[pl.ds(g * L, L)] = sb_idx[pl.ds(g * L, L)] + 1
    pltpu.sync_copy(rs.at[sb_idx], ebuf.at[pl.ds(0, 128)])

    pltpu.sync_copy(tab0.at[gidx], r0)
    pltpu.sync_copy(tab1.at[gidx], r1)

    @pl.loop(0, 128)
    def _accrows(b):
        for g in range(D // L):
            accbuf[b, pl.ds(g * L, L)] = (accbuf[b, pl.ds(g * L, L)]
                                          + r0[b, pl.ds(g * L, L)]
                                          + 2.0 * r1[b, pl.ds(g * L, L)])

    pltpu.sync_copy(zb, a2.at[pl.ds(sid * 128, 128)])

    @pl.loop(0, 128)
    def _build(b):
        s = _sget(sbuf, b)
        e = _sget(ebuf, b)
        g = b // 8
        col0 = (b % 8) * L
        m = _iota16() < (e - s)
        posb[g, pl.ds(col0, L)] = koff_e + s + _iota16()
        ridxb[g, pl.ds(col0, L)] = jnp.where(m, sid * 128 + b, TRASH)

    for g in range(SLOTS):
        pltpu.sync_copy(earr.at[posb.at[g]], sv)
        for gg in range(128 // L):
            gibuf[pl.ds(gg * L, L)] = sv[pl.ds(gg * L, L)] + koff_tab
        pltpu.sync_copy(gtab.at[gibuf], r0)
        pltpu.sync_copy(r0, a2.at[ridxb.at[g]], add=True)

    @pl.loop(0, 128)
    def _ovf(b):
        s = _sget(sbuf, b)
        e = _sget(ebuf, b)
        nov = (e - s - SLOTS + L - 1) // L

        @pl.loop(0, nov)
        def _chunk(t):
            base = s + SLOTS + t * L
            posv = koff_e + base + _iota16()
            m = (base + _iota16()) < e
            pltpu.sync_copy(earr.at[posv], sv16)
            giv = sv16[pl.ds(0, L)] + koff_tab
            pltpu.sync_copy(gtab.at[giv], rows16)
            ridxv = jnp.where(m, sid * 128 + b, TRASH)
            pltpu.sync_copy(rows16, a2.at[ridxv], add=True)

    pltpu.sync_copy(a2.at[pl.ds(sid * 128, 128)], r1)

    @pl.loop(0, 128)
    def _act(b):
        for g in range(D // L):
            v = r1[b, pl.ds(g * L, L)]
            accbuf[b, pl.ds(g * L, L)] = (accbuf[b, pl.ds(g * L, L)]
                                          + jnp.maximum(v, LEAKY * v))


def _final_body(U0f, I0f, U1f, I1f, adjs, tps, rsa, rst, uids, iids, preds,
                uv, ivv, gidx, sb_idx, sbuf, ebuf, sv, sv16, gibuf, posb, ridxb,
                r0, r1, rows16, fu, fi, zb, pb, a2):
    cid = lax.axis_index("c")
    sid = lax.axis_index("s")
    wloc = cid * NS + sid

    pltpu.sync_copy(uids.at[pl.ds(wloc * 128, 128)], uv)
    pltpu.sync_copy(iids.at[pl.ds(wloc * 128, 128)], ivv)

    @pl.loop(0, 128)
    def _zero(b):
        for g in range(D // L):
            fu[b, pl.ds(g * L, L)] = jnp.zeros((L,), jnp.float32)
            fi[b, pl.ds(g * L, L)] = jnp.zeros((L,), jnp.float32)
            zb[b, pl.ds(g * L, L)] = jnp.zeros((L,), jnp.float32)

    @pl.loop(0, G)
    def _graph(k):
        _demand_side(k, U0f, U1f, I1f, rsa, adjs, uv, fu,
                     gidx, sb_idx, sbuf, ebuf, sv, sv16, gibuf, posb, ridxb,
                     r0, r1, rows16, zb, a2, sid)
        _demand_side(k, I0f, I1f, U1f, rst, tps, ivv, fi,
                     gidx, sb_idx, sbuf, ebuf, sv, sv16, gibuf, posb, ridxb,
                     r0, r1, rows16, zb, a2, sid)

    @pl.loop(0, 128 // L)
    def _dot(bb):
        out = jnp.zeros((L,), jnp.float32)
        for j in range(L):
            b = bb * L + j
            part = fu[b, pl.ds(0, L)] * fi[b, pl.ds(0, L)]
            for g in range(1, D // L):
                part = part + fu[b, pl.ds(g * L, L)] * fi[b, pl.ds(g * L, L)]
            for s in (8, 4, 2, 1):
                part = part + jnp.take_along_axis(part, (_iota16() + s) % L, axis=0)
            out = jnp.where(_iota16() == j, part * (1.0 / 9.0), out)
        pb[pl.ds(bb * L, L)] = out

    pltpu.sync_copy(pb, preds.at[pl.ds(wloc * 128, 128)])


def kernel(user_embeddings, item_embeddings, adj_tgt, adj_src,
           tpadj_tgt, tpadj_src, uids, iids):
    f32 = jnp.float32
    U0f = user_embeddings.reshape(G * N, D)
    I0f = item_embeddings.reshape(G * N, D)

    adj_tgt = adj_tgt.astype(jnp.int32)
    adj_src = adj_src.astype(jnp.int32)
    tpadj_tgt = tpadj_tgt.astype(jnp.int32)
    tpadj_src = tpadj_src.astype(jnp.int32)
    uids32 = uids.astype(jnp.int32)
    iids32 = iids.astype(jnp.int32)

    def _offsets(tgt):
        deg = jnp.zeros((N,), jnp.int32).at[tgt].add(1, mode="drop")
        return jnp.concatenate([jnp.zeros((1,), jnp.int32), jnp.cumsum(deg)])

    rsa_f = jax.vmap(_offsets)(adj_tgt).astype(jnp.int32).reshape(-1)
    rst_f = jax.vmap(_offsets)(tpadj_tgt).astype(jnp.int32).reshape(-1)

    pad = ((0, 0), (0, EP - E))
    adjt_f = jnp.pad(adj_tgt, pad).reshape(-1)
    adjs_f = jnp.pad(adj_src, pad).reshape(-1)
    tpt_f = jnp.pad(tpadj_tgt, pad).reshape(-1)
    tps_f = jnp.pad(tpadj_src, pad).reshape(-1)

    mesh = plsc.VectorSubcoreMesh(core_axis_name="c", subcore_axis_name="s",
                                  num_cores=NC, num_subcores=NS)

    layer1 = pl.kernel(
        _layer1_body,
        out_type=[jax.ShapeDtypeStruct((G * N, D), f32),
                  jax.ShapeDtypeStruct((G * N, D), f32)],
        mesh=mesh,
        scratch_types=[
            pltpu.VMEM((2, KH, K), jnp.int32),
            pltpu.VMEM((2, KH, K), jnp.int32),
            pltpu.VMEM((2, KH, K), jnp.int32),
            pltpu.VMEM((2, KH, K), jnp.int32),
            pltpu.VMEM((2, K2, D), f32),
            pltpu.VMEM((RB, D), f32),
            pltpu.VMEM((RB, D), f32),
            pltpu.VMEM((RB, D), f32),
            pltpu.VMEM((2 * L,), jnp.int32),
            pltpu.VMEM_SHARED((C + 8, D), f32),
            pltpu.SemaphoreType.DMA((2,)),
            pltpu.SemaphoreType.DMA((2,)),
            pltpu.SemaphoreType.DMA((2,)),
            pltpu.SemaphoreType.DMA((2,)),
        ],
    )
    U1f, I1f = layer1(U0f, I0f, adjt_f, adjs_f, tpt_f, tps_f, rsa_f, rst_f)

    final = pl.kernel(
        _final_body,
        out_type=jax.ShapeDtypeStruct((B,), f32),
        mesh=mesh,
        scratch_types=[
            pltpu.VMEM((128,), jnp.int32),
            pltpu.VMEM((128,), jnp.int32),
            pltpu.VMEM((128,), jnp.int32),
            pltpu.VMEM((128,), jnp.int32),
            pltpu.VMEM((144,), jnp.int32),
            pltpu.VMEM((144,), jnp.int32),
            pltpu.VMEM((128,), jnp.int32),
            pltpu.VMEM((L,), jnp.int32),
            pltpu.VMEM((128,), jnp.int32),
            pltpu.VMEM((SLOTS, 128), jnp.int32),
            pltpu.VMEM((SLOTS, 128), jnp.int32),
            pltpu.VMEM((128, D), f32),
            pltpu.VMEM((128, D), f32),
            pltpu.VMEM((L, D), f32),
            pltpu.VMEM((128, D), f32),
            pltpu.VMEM((128, D), f32),
            pltpu.VMEM((128, D), f32),
            pltpu.VMEM((128,), f32),
            pltpu.VMEM_SHARED((TRASH + 8, D), f32),
        ],
    )
    return final(U0f, I0f, U1f, I1f, adjs_f, tps_f, rsa_f, rst_f,
                 uids32, iids32)


# speedup vs baseline: 1.1424x; 1.1047x over previous
import functools

import jax
import jax.numpy as jnp
from jax import lax
from jax.experimental import pallas as pl
from jax.experimental.pallas import tpu as pltpu
from jax.experimental.pallas import tpu_sc as plsc

G = 3
N = 50000
NP1 = N + 1
D = 128
E = 500000
EP = E + 512
B = 4096
LEAKY = 0.5

NC = 2
NS = 16
L = 16

C = 3200
NCHUNK = 16
CT = C // NS
RB = 40
K = 128
KH = 1
K2 = K * KH
SLOTS = 16
TRASH = NS * 128


def _iota16():
    return lax.iota(jnp.int32, L)


def _sget(ref, i):
    return ref[pl.ds(i, L)][0]


def _seg_pass(src_tab, tgt_arr, src_arr, rs, base_tab, out_tab,
              idxb, tgtb, sidxb, gixb, rowsb, epi_t, epi_b, zbuf, bnds,
              acc, semi, semt, semg, sems_, cid, sid):
    def _wait_idx(p):
        for h in range(KH):
            pltpu.make_async_copy(src_arr.at[pl.ds(0, K)], idxb.at[p, h],
                                  semi.at[p]).wait()
            pltpu.make_async_copy(tgt_arr.at[pl.ds(0, K)], tgtb.at[p, h],
                                  semt.at[p]).wait()

    def _wait_gather(p):
        pltpu.make_async_copy(src_tab.at[pl.ds(0, K2)], rowsb.at[p], semg.at[p]).wait()

    def _wait_scat(p):
        pltpu.make_async_copy(rowsb.at[p], acc.at[pl.ds(0, K2)], sems_.at[p]).wait()

    def _issue_gather(p):
        for h in range(KH):
            pltpu.async_copy(src_tab.at[gixb.at[p, h]],
                             rowsb.at[p, pl.ds(h * K, K)], semg.at[p])

    def _issue_scatter(p):
        for h in range(KH):
            pltpu.async_copy(rowsb.at[p, pl.ds(h * K, K)],
                             acc.at[sidxb.at[p, h]], sems_.at[p], add=True)

    @pl.loop(0, G)
    def _graph(k):
        koff_tab = k * N
        koff_e = k * EP
        koff_rs = k * NP1

        bidx = jnp.minimum(_iota16() * C, N) + koff_rs
        pltpu.sync_copy(rs.at[bidx], bnds.at[pl.ds(0, L)])
        bidx2 = jnp.minimum((_iota16() + L) * C, N) + koff_rs
        pltpu.sync_copy(rs.at[bidx2], bnds.at[pl.ds(L, L)])

        @pl.loop(0, NCHUNK // NC)
        def _chunk(j):
            c = NC * j + cid
            nlo = c * C
            nhi = jnp.minimum(nlo + C, N)

            @pl.loop(0, CT // RB)
            def _zero(r):
                pltpu.async_copy(zbuf, acc.at[pl.ds(sid * CT + r * RB, RB)],
                                 semg.at[0])

            pltpu.make_async_copy(epi_t, acc.at[pl.ds(0, CT)],
                                  semg.at[0]).wait()

            plsc.subcore_barrier()

            lo = _sget(bnds, c)
            hi = _sget(bnds, c + 1)
            ln = hi - lo
            el = lo + (ln * sid) // NS
            eh = lo + (ln * (sid + 1)) // NS
            el8 = (el // 8) * 8
            nsteps = (eh - el8 + K2 - 1) // K2

            def _fetch(t, p):
                base = el8 + t * K2
                for h in range(KH):
                    pltpu.async_copy(src_arr.at[pl.ds(koff_e + base + h * K, K)],
                                     idxb.at[p, h], semi.at[p])
                    pltpu.async_copy(tgt_arr.at[pl.ds(koff_e + base + h * K, K)],
                                     tgtb.at[p, h], semt.at[p])

            @pl.when(nsteps > 0)
            def _prologue():
                _fetch(0, 0)

            @pl.loop(0, nsteps)
            def _edges(t):
                p = t % 2
                q = 1 - p
                base = el8 + t * K2
                _wait_idx(p)

                @pl.when(t + 1 < nsteps)
                def _next():
                    _fetch(t + 1, q)

                @pl.when(t >= 2)
                def _reuse2():
                    _wait_scat(p)

                for g in range(K2 // L):
                    h = g // (K // L)
                    gl = g % (K // L)
                    tv = tgtb[p, h, pl.ds(gl * L, L)]
                    iv = idxb[p, h, pl.ds(gl * L, L)]
                    pos = base + g * L + _iota16()
                    m = (pos >= el) & (pos < eh)
                    sidxb[p, h, pl.ds(gl * L, L)] = jnp.where(m, tv - nlo, C)
                    gixb[p, h, pl.ds(gl * L, L)] = iv + koff_tab

                _issue_gather(p)

                @pl.when(t >= 1)
                def _pipe():
                    _wait_gather(q)
                    _issue_scatter(q)

            @pl.when(nsteps >= 1)
            def _tail():
                pl_ = (nsteps - 1) % 2
                _wait_gather(pl_)
                _issue_scatter(pl_)

                @pl.when(nsteps >= 2)
                def _d2():
                    _wait_scat(nsteps % 2)

                _wait_scat(pl_)

            plsc.subcore_barrier()

            rl = nlo + sid * CT
            cnt = jnp.minimum(rl + CT, nhi) - rl

            @pl.when(cnt > 0)
            def _epi():
                pltpu.async_copy(acc.at[pl.ds(sid * CT, CT)], epi_t, semi.at[0])
                pltpu.async_copy(base_tab.at[pl.ds(koff_tab + rl, CT)], epi_b,
                                 semt.at[0])
                pltpu.make_async_copy(acc.at[pl.ds(0, CT)], epi_t,
                                      semi.at[0]).wait()
                pltpu.make_async_copy(base_tab.at[pl.ds(0, CT)], epi_b,
                                      semt.at[0]).wait()

                @pl.loop(0, CT)
                def _row(rr):
                    for g in range(D // L):
                        t = epi_t[rr, pl.ds(g * L, L)]
                        bse = epi_b[rr, pl.ds(g * L, L)]
                        epi_t[rr, pl.ds(g * L, L)] = jnp.maximum(t, LEAKY * t) + bse

                pltpu.sync_copy(epi_t, out_tab.at[pl.ds(koff_tab + rl, CT)])

            plsc.subcore_barrier()


def _layer1_body(U0f, I0f, adjt, adjs, tpt, tps, rsa, rst, U1f, I1f,
                 idxb, tgtb, sidxb, gixb, rowsb, epi_t, epi_b, zbuf, bnds, acc,
                 semi, semt, semg, sems_):
    cid = lax.axis_index("c")
    sid = lax.axis_index("s")

    @pl.loop(0, RB)
    def _z(rr):
        for g in range(D // L):
            zbuf[rr, pl.ds(g * L, L)] = jnp.zeros((L,), jnp.float32)

    _seg_pass(I0f, adjt, adjs, rsa, U0f, U1f,
              idxb, tgtb, sidxb, gixb, rowsb, epi_t, epi_b, zbuf, bnds, acc,
              semi, semt, semg, sems_, cid, sid)
    _seg_pass(U0f, tpt, tps, rst, I0f, I1f,
              idxb, tgtb, sidxb, gixb, rowsb, epi_t, epi_b, zbuf, bnds, acc,
              semi, semt, semg, sems_, cid, sid)


def _demand_side(k, tab0, tab1, gtab, rs, earr, ids, accbuf,
                 gidx, sb_idx, sbuf, ebuf, sv, sv16, gibuf, posb, ridxb,
                 r0, r1, rows16, zb, a2, sid):
    koff_tab = k * N
    koff_e = k * EP
    koff_rs = k * NP1

    for g in range(128 // L):
        iv = ids[pl.ds(g * L, L)]
        gidx[pl.ds(g * L, L)] = iv + koff_tab
        sb_idx[pl.ds(g * L, L)] = iv + koff_rs
    pltpu.sync_copy(rs.at[sb_idx], sbuf.at[pl.ds(0, 128)])
    for g in range(128 // L):
        sb_idx[pl.ds(g * L, L)] = sb_idx[pl.ds(g * L, L)] + 1
    pltpu.sync_copy(rs.at[sb_idx], ebuf.at[pl.ds(0, 128)])

    pltpu.sync_copy(tab0.at[gidx], r0)
    pltpu.sync_copy(tab1.at[gidx], r1)

    @pl.loop(0, 128)
    def _accrows(b):
        for g in range(D // L):
            accbuf[b, pl.ds(g * L, L)] = (accbuf[b, pl.ds(g * L, L)]
                                          + r0[b, pl.ds(g * L, L)]
                                          + 2.0 * r1[b, pl.ds(g * L, L)])

    pltpu.sync_copy(zb, a2.at[pl.ds(sid * 128, 128)])

    @pl.loop(0, 128)
    def _build(b):
        s = _sget(sbuf, b)
        e = _sget(ebuf, b)
        g = b // 8
        col0 = (b % 8) * L
        m = _iota16() < (e - s)
        posb[g, pl.ds(col0, L)] = koff_e + s + _iota16()
        ridxb[g, pl.ds(col0, L)] = jnp.where(m, sid * 128 + b, TRASH)

    for g in range(SLOTS):
        pltpu.sync_copy(earr.at[posb.at[g]], sv)
        for gg in range(128 // L):
            gibuf[pl.ds(gg * L, L)] = sv[pl.ds(gg * L, L)] + koff_tab
        pltpu.sync_copy(gtab.at[gibuf], r0)
        pltpu.sync_copy(r0, a2.at[ridxb.at[g]], add=True)

    @pl.loop(0, 128)
    def _ovf(b):
        s = _sget(sbuf, b)
        e = _sget(ebuf, b)
        nov = (e - s - SLOTS + L - 1) // L

        @pl.loop(0, nov)
        def _chunk(t):
            base = s + SLOTS + t * L
            posv = koff_e + base + _iota16()
            m = (base + _iota16()) < e
            pltpu.sync_copy(earr.at[posv], sv16)
            giv = sv16[pl.ds(0, L)] + koff_tab
            pltpu.sync_copy(gtab.at[giv], rows16)
            ridxv = jnp.where(m, sid * 128 + b, TRASH)
            pltpu.sync_copy(rows16, a2.at[ridxv], add=True)

    pltpu.sync_copy(a2.at[pl.ds(sid * 128, 128)], r1)

    @pl.loop(0, 128)
    def _act(b):
        for g in range(D // L):
            v = r1[b, pl.ds(g * L, L)]
            accbuf[b, pl.ds(g * L, L)] = (accbuf[b, pl.ds(g * L, L)]
                                          + jnp.maximum(v, LEAKY * v))


def _final_body(U0f, I0f, U1f, I1f, adjs, tps, rsa, rst, uids, iids, preds,
                uv, ivv, gidx, sb_idx, sbuf, ebuf, sv, sv16, gibuf, posb, ridxb,
                r0, r1, rows16, fu, fi, zb, pb, a2):
    cid = lax.axis_index("c")
    sid = lax.axis_index("s")
    wloc = cid * NS + sid

    pltpu.sync_copy(uids.at[pl.ds(wloc * 128, 128)], uv)
    pltpu.sync_copy(iids.at[pl.ds(wloc * 128, 128)], ivv)

    @pl.loop(0, 128)
    def _zero(b):
        for g in range(D // L):
            fu[b, pl.ds(g * L, L)] = jnp.zeros((L,), jnp.float32)
            fi[b, pl.ds(g * L, L)] = jnp.zeros((L,), jnp.float32)
            zb[b, pl.ds(g * L, L)] = jnp.zeros((L,), jnp.float32)

    @pl.loop(0, G)
    def _graph(k):
        _demand_side(k, U0f, U1f, I1f, rsa, adjs, uv, fu,
                     gidx, sb_idx, sbuf, ebuf, sv, sv16, gibuf, posb, ridxb,
                     r0, r1, rows16, zb, a2, sid)
        _demand_side(k, I0f, I1f, U1f, rst, tps, ivv, fi,
                     gidx, sb_idx, sbuf, ebuf, sv, sv16, gibuf, posb, ridxb,
                     r0, r1, rows16, zb, a2, sid)

    @pl.loop(0, 128 // L)
    def _dot(bb):
        out = jnp.zeros((L,), jnp.float32)
        for j in range(L):
            b = bb * L + j
            part = fu[b, pl.ds(0, L)] * fi[b, pl.ds(0, L)]
            for g in range(1, D // L):
                part = part + fu[b, pl.ds(g * L, L)] * fi[b, pl.ds(g * L, L)]
            for s in (8, 4, 2, 1):
                part = part + jnp.take_along_axis(part, (_iota16() + s) % L, axis=0)
            out = jnp.where(_iota16() == j, part * (1.0 / 9.0), out)
        pb[pl.ds(bb * L, L)] = out

    pltpu.sync_copy(pb, preds.at[pl.ds(wloc * 128, 128)])


def kernel(user_embeddings, item_embeddings, adj_tgt, adj_src,
           tpadj_tgt, tpadj_src, uids, iids):
    f32 = jnp.float32
    U0f = user_embeddings.reshape(G * N, D)
    I0f = item_embeddings.reshape(G * N, D)

    adj_tgt = adj_tgt.astype(jnp.int32)
    adj_src = adj_src.astype(jnp.int32)
    tpadj_tgt = tpadj_tgt.astype(jnp.int32)
    tpadj_src = tpadj_src.astype(jnp.int32)
    uids32 = uids.astype(jnp.int32)
    iids32 = iids.astype(jnp.int32)

    def _offsets(tgt):
        deg = jnp.zeros((N,), jnp.int32).at[tgt].add(1, mode="drop")
        return jnp.concatenate([jnp.zeros((1,), jnp.int32), jnp.cumsum(deg)])

    rsa_f = jax.vmap(_offsets)(adj_tgt).astype(jnp.int32).reshape(-1)
    rst_f = jax.vmap(_offsets)(tpadj_tgt).astype(jnp.int32).reshape(-1)

    pad = ((0, 0), (0, EP - E))
    adjt_f = jnp.pad(adj_tgt, pad).reshape(-1)
    adjs_f = jnp.pad(adj_src, pad).reshape(-1)
    tpt_f = jnp.pad(tpadj_tgt, pad).reshape(-1)
    tps_f = jnp.pad(tpadj_src, pad).reshape(-1)

    mesh = plsc.VectorSubcoreMesh(core_axis_name="c", subcore_axis_name="s",
                                  num_cores=NC, num_subcores=NS)

    layer1 = pl.kernel(
        _layer1_body,
        out_type=[jax.ShapeDtypeStruct((G * N, D), f32),
                  jax.ShapeDtypeStruct((G * N, D), f32)],
        mesh=mesh,
        scratch_types=[
            pltpu.VMEM((2, KH, K), jnp.int32),
            pltpu.VMEM((2, KH, K), jnp.int32),
            pltpu.VMEM((2, KH, K), jnp.int32),
            pltpu.VMEM((2, KH, K), jnp.int32),
            pltpu.VMEM((2, K2, D), f32),
            pltpu.VMEM((CT, D), f32),
            pltpu.VMEM((CT, D), f32),
            pltpu.VMEM((RB, D), f32),
            pltpu.VMEM((2 * L,), jnp.int32),
            pltpu.VMEM_SHARED((C + 8, D), f32),
            pltpu.SemaphoreType.DMA((2,)),
            pltpu.SemaphoreType.DMA((2,)),
            pltpu.SemaphoreType.DMA((2,)),
            pltpu.SemaphoreType.DMA((2,)),
        ],
    )
    U1f, I1f = layer1(U0f, I0f, adjt_f, adjs_f, tpt_f, tps_f, rsa_f, rst_f)

    final = pl.kernel(
        _final_body,
        out_type=jax.ShapeDtypeStruct((B,), f32),
        mesh=mesh,
        scratch_types=[
            pltpu.VMEM((128,), jnp.int32),
            pltpu.VMEM((128,), jnp.int32),
            pltpu.VMEM((128,), jnp.int32),
            pltpu.VMEM((128,), jnp.int32),
            pltpu.VMEM((144,), jnp.int32),
            pltpu.VMEM((144,), jnp.int32),
            pltpu.VMEM((128,), jnp.int32),
            pltpu.VMEM((L,), jnp.int32),
            pltpu.VMEM((128,), jnp.int32),
            pltpu.VMEM((SLOTS, 128), jnp.int32),
            pltpu.VMEM((SLOTS, 128), jnp.int32),
            pltpu.VMEM((128, D), f32),
            pltpu.VMEM((128, D), f32),
            pltpu.VMEM((L, D), f32),
            pltpu.VMEM((128, D), f32),
            pltpu.VMEM((128, D), f32),
            pltpu.VMEM((128, D), f32),
            pltpu.VMEM((128,), f32),
            pltpu.VMEM_SHARED((TRASH + 8, D), f32),
        ],
    )
    return final(U0f, I0f, U1f, I1f, adjs_f, tps_f, rsa_f, rst_f,
                 uids32, iids32)


# speedup vs baseline: 1.1699x; 1.0241x over previous
import functools

import jax
import jax.numpy as jnp
from jax import lax
from jax.experimental import pallas as pl
from jax.experimental.pallas import tpu as pltpu
from jax.experimental.pallas import tpu_sc as plsc

G = 3
N = 50000
NP1 = N + 1
D = 128
E = 500000
EP = E + 512
B = 4096
LEAKY = 0.5

NC = 2
NS = 16
L = 16

C = 3200
NCHUNK = 16
CT = C // NS
RB = 40
K = 128
KH = 1
K2 = K * KH
SLOTS = 16
TRASH = NS * 128


def _iota16():
    return lax.iota(jnp.int32, L)


def _sget(ref, i):
    return ref[pl.ds(i, L)][0]


def _seg_pass(src_tab, tgt_arr, src_arr, rs, base_tab, out_tab,
              idxb, tgtb, sidxb, gixb, rowsb, epi_t, epi_b, zbuf, bnds,
              acc, semi, semt, semg, sems_, cid, sid):
    def _wait_idx(p):
        for h in range(KH):
            pltpu.make_async_copy(src_arr.at[pl.ds(0, K)], idxb.at[p, h],
                                  semi.at[p]).wait()
            pltpu.make_async_copy(tgt_arr.at[pl.ds(0, K)], tgtb.at[p, h],
                                  semt.at[p]).wait()

    def _wait_gather(p):
        pltpu.make_async_copy(src_tab.at[pl.ds(0, K2)], rowsb.at[p], semg.at[p]).wait()

    def _wait_scat(p):
        pltpu.make_async_copy(rowsb.at[p], acc.at[pl.ds(0, K2)], sems_.at[p]).wait()

    def _issue_gather(p):
        for h in range(KH):
            pltpu.async_copy(src_tab.at[gixb.at[p, h]],
                             rowsb.at[p, pl.ds(h * K, K)], semg.at[p])

    def _issue_scatter(p):
        for h in range(KH):
            pltpu.async_copy(rowsb.at[p, pl.ds(h * K, K)],
                             acc.at[sidxb.at[p, h]], sems_.at[p], add=True)

    @pl.loop(0, G)
    def _graph(k):
        koff_tab = k * N
        koff_e = k * EP
        koff_rs = k * NP1

        bidx = jnp.minimum(_iota16() * C, N) + koff_rs
        pltpu.sync_copy(rs.at[bidx], bnds.at[pl.ds(0, L)])
        bidx2 = jnp.minimum((_iota16() + L) * C, N) + koff_rs
        pltpu.sync_copy(rs.at[bidx2], bnds.at[pl.ds(L, L)])

        @pl.loop(0, NCHUNK // NC)
        def _chunk(j):
            c = NC * j + cid
            nlo = c * C
            nhi = jnp.minimum(nlo + C, N)

            @pl.loop(0, CT // RB)
            def _zero(r):
                pltpu.async_copy(zbuf, acc.at[pl.ds(sid * CT + r * RB, RB)],
                                 semg.at[0])

            pltpu.make_async_copy(epi_t, acc.at[pl.ds(0, CT)],
                                  semg.at[0]).wait()

            plsc.subcore_barrier()

            lo = _sget(bnds, c)
            hi = _sget(bnds, c + 1)
            ln = hi - lo
            el = lo + (ln * sid) // NS
            eh = lo + (ln * (sid + 1)) // NS
            el8 = (el // 8) * 8
            nsteps = (eh - el8 + K2 - 1) // K2

            def _fetch(t, p):
                base = el8 + t * K2
                for h in range(KH):
                    pltpu.async_copy(src_arr.at[pl.ds(koff_e + base + h * K, K)],
                                     idxb.at[p, h], semi.at[p])
                    pltpu.async_copy(tgt_arr.at[pl.ds(koff_e + base + h * K, K)],
                                     tgtb.at[p, h], semt.at[p])

            @pl.when(nsteps > 0)
            def _prologue():
                _fetch(0, 0)

            @pl.loop(0, nsteps)
            def _edges(t):
                p = t % 2
                q = 1 - p
                base = el8 + t * K2
                _wait_idx(p)

                @pl.when(t + 1 < nsteps)
                def _next():
                    _fetch(t + 1, q)

                @pl.when(t >= 2)
                def _reuse2():
                    _wait_scat(p)

                for g in range(K2 // L):
                    h = g // (K // L)
                    gl = g % (K // L)
                    tv = tgtb[p, h, pl.ds(gl * L, L)]
                    iv = idxb[p, h, pl.ds(gl * L, L)]
                    pos = base + g * L + _iota16()
                    m = (pos >= el) & (pos < eh)
                    sidxb[p, h, pl.ds(gl * L, L)] = jnp.where(m, tv - nlo, C)
                    gixb[p, h, pl.ds(gl * L, L)] = iv + koff_tab

                _issue_gather(p)

                @pl.when(t >= 1)
                def _pipe():
                    _wait_gather(q)
                    _issue_scatter(q)

            @pl.when(nsteps >= 1)
            def _tail():
                pl_ = (nsteps - 1) % 2
                _wait_gather(pl_)
                _issue_scatter(pl_)

                @pl.when(nsteps >= 2)
                def _d2():
                    _wait_scat(nsteps % 2)

                _wait_scat(pl_)

            plsc.subcore_barrier()

            rl = nlo + sid * CT
            cnt = jnp.minimum(rl + CT, nhi) - rl

            @pl.when(cnt > 0)
            def _epi():
                pltpu.async_copy(acc.at[pl.ds(sid * CT, CT)], epi_t, semi.at[0])
                pltpu.async_copy(base_tab.at[pl.ds(koff_tab + rl, CT)], epi_b,
                                 semt.at[0])
                pltpu.make_async_copy(acc.at[pl.ds(0, CT)], epi_t,
                                      semi.at[0]).wait()
                pltpu.make_async_copy(base_tab.at[pl.ds(0, CT)], epi_b,
                                      semt.at[0]).wait()

                @pl.loop(0, CT)
                def _row(rr):
                    for g in range(D // L):
                        t = epi_t[rr, pl.ds(g * L, L)]
                        bse = epi_b[rr, pl.ds(g * L, L)]
                        epi_t[rr, pl.ds(g * L, L)] = jnp.maximum(t, LEAKY * t) + bse

                pltpu.sync_copy(epi_t, out_tab.at[pl.ds(koff_tab + rl, CT)])

            plsc.subcore_barrier()


def _layer1_body(U0f, I0f, adjt, adjs, tpt, tps, rsa, rst, U1f, I1f,
                 idxb, tgtb, sidxb, gixb, rowsb, epi_t, epi_b, zbuf, bnds, acc,
                 semi, semt, semg, sems_):
    cid = lax.axis_index("c")
    sid = lax.axis_index("s")

    @pl.loop(0, RB)
    def _z(rr):
        for g in range(D // L):
            zbuf[rr, pl.ds(g * L, L)] = jnp.zeros((L,), jnp.float32)

    _seg_pass(I0f, adjt, adjs, rsa, U0f, U1f,
              idxb, tgtb, sidxb, gixb, rowsb, epi_t, epi_b, zbuf, bnds, acc,
              semi, semt, semg, sems_, cid, sid)
    _seg_pass(U0f, tpt, tps, rst, I0f, I1f,
              idxb, tgtb, sidxb, gixb, rowsb, epi_t, epi_b, zbuf, bnds, acc,
              semi, semt, semg, sems_, cid, sid)


def _demand_side(k, tab0, tab1, gtab, rs, earr, ids, accbuf,
                 gidx, sb_idx, sbuf, ebuf, svb, sv16, gib2, posb, ridxb,
                 r0, r1, rows16, zb, a2, bsi, bsg, bss, sid):
    koff_tab = k * N
    koff_e = k * EP
    koff_rs = k * NP1

    for g in range(128 // L):
        iv = ids[pl.ds(g * L, L)]
        gidx[pl.ds(g * L, L)] = iv + koff_tab
        sb_idx[pl.ds(g * L, L)] = iv + koff_rs
    pltpu.sync_copy(rs.at[sb_idx], sbuf.at[pl.ds(0, 128)])
    for g in range(128 // L):
        sb_idx[pl.ds(g * L, L)] = sb_idx[pl.ds(g * L, L)] + 1
    pltpu.sync_copy(rs.at[sb_idx], ebuf.at[pl.ds(0, 128)])

    pltpu.sync_copy(tab0.at[gidx], r0)
    pltpu.sync_copy(tab1.at[gidx], r1)

    @pl.loop(0, 128)
    def _accrows(b):
        for g in range(D // L):
            accbuf[b, pl.ds(g * L, L)] = (accbuf[b, pl.ds(g * L, L)]
                                          + r0[b, pl.ds(g * L, L)]
                                          + 2.0 * r1[b, pl.ds(g * L, L)])

    pltpu.sync_copy(zb, a2.at[pl.ds(sid * 128, 128)])

    @pl.loop(0, 128)
    def _build(b):
        s = _sget(sbuf, b)
        e = _sget(ebuf, b)
        g = b // 8
        col0 = (b % 8) * L
        m = _iota16() < (e - s)
        posb[g, pl.ds(col0, L)] = koff_e + s + _iota16()
        ridxb[g, pl.ds(col0, L)] = jnp.where(m, sid * 128 + b, TRASH)

    def _svwait(pg):
        pltpu.make_async_copy(earr.at[pl.ds(0, 128)], svb.at[pg],
                              bsi.at[pg]).wait()

    def _rwait(pg):
        rbuf = r0 if pg == 0 else r1
        pltpu.make_async_copy(gtab.at[pl.ds(0, 128)], rbuf, bsg.at[pg]).wait()

    def _swait(pg):
        rbuf = r0 if pg == 0 else r1
        pltpu.make_async_copy(rbuf, a2.at[pl.ds(0, 128)], bss.at[pg]).wait()

    pltpu.async_copy(earr.at[posb.at[0]], svb.at[0], bsi.at[0])
    for g in range(SLOTS):
        pg = g % 2
        qg = 1 - pg
        _svwait(pg)
        if g + 1 < SLOTS:
            pltpu.async_copy(earr.at[posb.at[g + 1]], svb.at[qg], bsi.at[qg])
        if g >= 2:
            _swait(pg)
        for gg in range(128 // L):
            gib2[pg, pl.ds(gg * L, L)] = svb[pg, pl.ds(gg * L, L)] + koff_tab
        rbuf = r0 if pg == 0 else r1
        pltpu.async_copy(gtab.at[gib2.at[pg]], rbuf, bsg.at[pg])
        if g >= 1:
            _rwait(qg)
            qbuf = r0 if qg == 0 else r1
            pltpu.async_copy(qbuf, a2.at[ridxb.at[g - 1]], bss.at[qg], add=True)
    lastp = (SLOTS - 1) % 2
    _rwait(lastp)
    lastbuf = r0 if lastp == 0 else r1
    pltpu.async_copy(lastbuf, a2.at[ridxb.at[SLOTS - 1]], bss.at[lastp], add=True)
    _swait(SLOTS % 2)
    _swait(lastp)

    @pl.loop(0, 128)
    def _ovf(b):
        s = _sget(sbuf, b)
        e = _sget(ebuf, b)
        nov = (e - s - SLOTS + L - 1) // L

        @pl.loop(0, nov)
        def _chunk(t):
            base = s + SLOTS + t * L
            posv = koff_e + base + _iota16()
            m = (base + _iota16()) < e
            pltpu.sync_copy(earr.at[posv], sv16)
            giv = sv16[pl.ds(0, L)] + koff_tab
            pltpu.sync_copy(gtab.at[giv], rows16)
            ridxv = jnp.where(m, sid * 128 + b, TRASH)
            pltpu.sync_copy(rows16, a2.at[ridxv], add=True)

    pltpu.sync_copy(a2.at[pl.ds(sid * 128, 128)], r1)

    @pl.loop(0, 128)
    def _act(b):
        for g in range(D // L):
            v = r1[b, pl.ds(g * L, L)]
            accbuf[b, pl.ds(g * L, L)] = (accbuf[b, pl.ds(g * L, L)]
                                          + jnp.maximum(v, LEAKY * v))


def _final_body(U0f, I0f, U1f, I1f, adjs, tps, rsa, rst, uids, iids, preds,
                uv, ivv, gidx, sb_idx, sbuf, ebuf, svb, sv16, gib2, posb, ridxb,
                r0, r1, rows16, fu, fi, zb, pb, a2, bsi, bsg, bss):
    cid = lax.axis_index("c")
    sid = lax.axis_index("s")
    wloc = cid * NS + sid

    pltpu.sync_copy(uids.at[pl.ds(wloc * 128, 128)], uv)
    pltpu.sync_copy(iids.at[pl.ds(wloc * 128, 128)], ivv)

    @pl.loop(0, 128)
    def _zero(b):
        for g in range(D // L):
            fu[b, pl.ds(g * L, L)] = jnp.zeros((L,), jnp.float32)
            fi[b, pl.ds(g * L, L)] = jnp.zeros((L,), jnp.float32)
            zb[b, pl.ds(g * L, L)] = jnp.zeros((L,), jnp.float32)

    @pl.loop(0, G)
    def _graph(k):
        _demand_side(k, U0f, U1f, I1f, rsa, adjs, uv, fu,
                     gidx, sb_idx, sbuf, ebuf, svb, sv16, gib2, posb, ridxb,
                     r0, r1, rows16, zb, a2, bsi, bsg, bss, sid)
        _demand_side(k, I0f, I1f, U1f, rst, tps, ivv, fi,
                     gidx, sb_idx, sbuf, ebuf, svb, sv16, gib2, posb, ridxb,
                     r0, r1, rows16, zb, a2, bsi, bsg, bss, sid)

    @pl.loop(0, 128 // L)
    def _dot(bb):
        out = jnp.zeros((L,), jnp.float32)
        for j in range(L):
            b = bb * L + j
            part = fu[b, pl.ds(0, L)] * fi[b, pl.ds(0, L)]
            for g in range(1, D // L):
                part = part + fu[b, pl.ds(g * L, L)] * fi[b, pl.ds(g * L, L)]
            for s in (8, 4, 2, 1):
                part = part + jnp.take_along_axis(part, (_iota16() + s) % L, axis=0)
            out = jnp.where(_iota16() == j, part * (1.0 / 9.0), out)
        pb[pl.ds(bb * L, L)] = out

    pltpu.sync_copy(pb, preds.at[pl.ds(wloc * 128, 128)])


def kernel(user_embeddings, item_embeddings, adj_tgt, adj_src,
           tpadj_tgt, tpadj_src, uids, iids):
    f32 = jnp.float32
    U0f = user_embeddings.reshape(G * N, D)
    I0f = item_embeddings.reshape(G * N, D)

    adj_tgt = adj_tgt.astype(jnp.int32)
    adj_src = adj_src.astype(jnp.int32)
    tpadj_tgt = tpadj_tgt.astype(jnp.int32)
    tpadj_src = tpadj_src.astype(jnp.int32)
    uids32 = uids.astype(jnp.int32)
    iids32 = iids.astype(jnp.int32)

    tgts = jnp.concatenate([adj_tgt, tpadj_tgt], axis=0)
    toff = (jnp.arange(2 * G, dtype=jnp.int32) * N)[:, None]
    flat = (tgts + toff).reshape(-1)
    deg = jnp.zeros((2 * G * N,), jnp.int32).at[flat].add(1, mode="drop")
    cs = jnp.cumsum(deg.reshape(2 * G, N), axis=-1)
    rs_all = jnp.concatenate([jnp.zeros((2 * G, 1), jnp.int32), cs], axis=-1)
    rsa_f = rs_all[:G].reshape(-1)
    rst_f = rs_all[G:].reshape(-1)

    pad = ((0, 0), (0, EP - E))
    adjt_f = jnp.pad(adj_tgt, pad).reshape(-1)
    adjs_f = jnp.pad(adj_src, pad).reshape(-1)
    tpt_f = jnp.pad(tpadj_tgt, pad).reshape(-1)
    tps_f = jnp.pad(tpadj_src, pad).reshape(-1)

    mesh = plsc.VectorSubcoreMesh(core_axis_name="c", subcore_axis_name="s",
                                  num_cores=NC, num_subcores=NS)

    layer1 = pl.kernel(
        _layer1_body,
        out_type=[jax.ShapeDtypeStruct((G * N, D), f32),
                  jax.ShapeDtypeStruct((G * N, D), f32)],
        mesh=mesh,
        scratch_types=[
            pltpu.VMEM((2, KH, K), jnp.int32),
            pltpu.VMEM((2, KH, K), jnp.int32),
            pltpu.VMEM((2, KH, K), jnp.int32),
            pltpu.VMEM((2, KH, K), jnp.int32),
            pltpu.VMEM((2, K2, D), f32),
            pltpu.VMEM((CT, D), f32),
            pltpu.VMEM((CT, D), f32),
            pltpu.VMEM((RB, D), f32),
            pltpu.VMEM((2 * L,), jnp.int32),
            pltpu.VMEM_SHARED((C + 8, D), f32),
            pltpu.SemaphoreType.DMA((2,)),
            pltpu.SemaphoreType.DMA((2,)),
            pltpu.SemaphoreType.DMA((2,)),
            pltpu.SemaphoreType.DMA((2,)),
        ],
    )
    U1f, I1f = layer1(U0f, I0f, adjt_f, adjs_f, tpt_f, tps_f, rsa_f, rst_f)

    final = pl.kernel(
        _final_body,
        out_type=jax.ShapeDtypeStruct((B,), f32),
        mesh=mesh,
        scratch_types=[
            pltpu.VMEM((128,), jnp.int32),
            pltpu.VMEM((128,), jnp.int32),
            pltpu.VMEM((128,), jnp.int32),
            pltpu.VMEM((128,), jnp.int32),
            pltpu.VMEM((144,), jnp.int32),
            pltpu.VMEM((144,), jnp.int32),
            pltpu.VMEM((2, 128), jnp.int32),
            pltpu.VMEM((L,), jnp.int32),
            pltpu.VMEM((2, 128), jnp.int32),
            pltpu.VMEM((SLOTS, 128), jnp.int32),
            pltpu.VMEM((SLOTS, 128), jnp.int32),
            pltpu.VMEM((128, D), f32),
            pltpu.VMEM((128, D), f32),
            pltpu.VMEM((L, D), f32),
            pltpu.VMEM((128, D), f32),
            pltpu.VMEM((128, D), f32),
            pltpu.VMEM((128, D), f32),
            pltpu.VMEM((128,), f32),
            pltpu.VMEM_SHARED((TRASH + 8, D), f32),
            pltpu.SemaphoreType.DMA((2,)),
            pltpu.SemaphoreType.DMA((2,)),
            pltpu.SemaphoreType.DMA((2,)),
        ],
    )
    return final(U0f, I0f, U1f, I1f, adjs_f, tps_f, rsa_f, rst_f,
                 uids32, iids32)
